# baseline (speedup 1.0000x reference)
import functools

import jax
import jax.numpy as jnp
from jax.experimental import pallas as pl
from jax.experimental.pallas import tpu as pltpu

B, N, D_IN, D_H, D_OUT, K = 32, 512, 256, 256, 128, 16


def _dot(a, b):
    return jax.lax.dot_general(
        a, b, (((1,), (0,)), ((), ())),
        preferred_element_type=jnp.float32,
    )


def _bdot(a, b):
    return jax.lax.dot_general(
        a.astype(jnp.bfloat16), b.astype(jnp.bfloat16),
        (((1,), (0,)), ((), ())),
        preferred_element_type=jnp.float32,
    )


def _gcntn_kernel(x1_ref, x2_ref, l1_ref, l2_ref, w1_ref, w2_ref, wt_ref,
                  v_ref, b_ref, wo_ref, out_ref):
    w1 = w1_ref[...]
    w2 = w2_ref[...]

    def tower(x_ref, l_ref):
        x = x_ref[0]
        l = l_ref[0]
        h = jnp.maximum(_bdot(l, _bdot(x, w1)), 0.0)
        h = jnp.maximum(_bdot(l, _bdot(h, w2)), 0.0)
        return jnp.mean(h, axis=0, keepdims=True)

    e1 = tower(x1_ref, l1_ref)
    e2 = tower(x2_ref, l2_ref)

    wt = wt_ref[...].reshape(K * D_OUT, D_OUT)
    tmp = _dot(wt, e2.reshape(D_OUT, 1)).reshape(K, D_OUT)
    bil = _dot(tmp, e1.reshape(D_OUT, 1))

    v = v_ref[...]
    lin = (_dot(v[:, :D_OUT], e1.reshape(D_OUT, 1))
           + _dot(v[:, D_OUT:], e2.reshape(D_OUT, 1)))

    ntn = jnp.maximum(bil + lin + b_ref[...].reshape(K, 1), 0.0)
    out_ref[0] = jnp.sum(ntn * wo_ref[...], axis=(0, 1), keepdims=True)


@jax.jit
def kernel(inputs_1, inputs_2, laplacians_1, laplacians_2, W1, W2, Wt, V,
           b_ntn, w_out):
    full = lambda *shape: pl.BlockSpec(shape, lambda b: (0,) * len(shape))
    batched = lambda *shape: pl.BlockSpec((1,) + shape,
                                          lambda b: (b,) + (0,) * len(shape))
    out = pl.pallas_call(
        _gcntn_kernel,
        grid=(B,),
        in_specs=[
            batched(N, D_IN), batched(N, D_IN),
            batched(N, N), batched(N, N),
            full(D_IN, D_H), full(D_H, D_OUT),
            full(K, D_OUT, D_OUT), full(K, 2 * D_OUT),
            full(1, K), full(K, 1),
        ],
        out_specs=pl.BlockSpec((1, 1, 1), lambda b: (b, 0, 0)),
        out_shape=jax.ShapeDtypeStruct((B, 1, 1), jnp.float32),
        compiler_params=pltpu.CompilerParams(
            dimension_semantics=("parallel",),
        ),
    )(inputs_1, inputs_2, laplacians_1, laplacians_2, W1, W2, Wt, V,
      b_ntn.reshape(1, K), w_out)
    return out[:, 0, 0]


# speedup vs baseline: 1.0039x; 1.0039x over previous
import jax
import jax.numpy as jnp
from jax.experimental import pallas as pl
from jax.experimental.pallas import tpu as pltpu

B, N, D_IN, D_H, D_OUT, K = 32, 512, 256, 256, 128, 16


def _dot(a, b):
    return jax.lax.dot_general(
        a, b, (((1,), (0,)), ((), ())),
        preferred_element_type=jnp.float32,
    )


def _gcntn_kernel(x1_ref, x2_ref, l1_ref, l2_ref, w1_ref, w2_ref, wtr_ref,
                  seg_ref, v1t_ref, v2t_ref, b_ref, wo_ref, out_ref, e_ref):
    b = pl.program_id(0)
    w1 = w1_ref[...]
    w2 = w2_ref[...]
    pool = jnp.full((1, N), 1.0 / N, dtype=jnp.float32)

    def tower(x_ref, l_ref, row):
        x = x_ref[0]
        l = l_ref[0]
        h = jnp.maximum(_dot(l, _dot(x, w1)), 0.0)
        h = jnp.maximum(_dot(l, _dot(h, w2)), 0.0)
        e_ref[pl.ds(row, 1), :] = _dot(pool, h)

    tower(x1_ref, l1_ref, b)
    tower(x2_ref, l2_ref, b + B)

    @pl.when(b == B - 1)
    def _ntn():
        e1 = e_ref[0:B, :]
        e2 = e_ref[B:2 * B, :]
        t = _dot(e1, wtr_ref[...])
        bil = _dot(t * jnp.tile(e2, (1, K)), seg_ref[...])
        lin = _dot(e1, v1t_ref[...]) + _dot(e2, v2t_ref[...])
        ntn = jnp.maximum(bil + lin + b_ref[...], 0.0)
        out_ref[...] = _dot(ntn, wo_ref[...])


@jax.jit
def kernel(inputs_1, inputs_2, laplacians_1, laplacians_2, W1, W2, Wt, V,
           b_ntn, w_out):
    wt_r = jnp.transpose(Wt, (1, 0, 2)).reshape(D_OUT, K * D_OUT)
    seg = (jnp.arange(K * D_OUT)[:, None] // D_OUT
           == jnp.arange(K)[None, :]).astype(jnp.float32)
    v_t = V.T

    full = lambda *shape: pl.BlockSpec(shape, lambda b: (0,) * len(shape))
    batched = lambda *shape: pl.BlockSpec((1,) + shape,
                                          lambda b: (b,) + (0,) * len(shape))
    out = pl.pallas_call(
        _gcntn_kernel,
        grid=(B,),
        in_specs=[
            batched(N, D_IN), batched(N, D_IN),
            batched(N, N), batched(N, N),
            full(D_IN, D_H), full(D_H, D_OUT),
            full(D_OUT, K * D_OUT), full(K * D_OUT, K),
            full(D_OUT, K), full(D_OUT, K),
            full(1, K), full(K, 1),
        ],
        out_specs=pl.BlockSpec((B, 1), lambda b: (0, 0)),
        out_shape=jax.ShapeDtypeStruct((B, 1), jnp.float32),
        scratch_shapes=[pltpu.VMEM((2 * B, D_OUT), jnp.float32)],
        compiler_params=pltpu.CompilerParams(
            dimension_semantics=("arbitrary",),
        ),
    )(inputs_1, inputs_2, laplacians_1, laplacians_2, W1, W2, wt_r, seg,
      v_t[:D_OUT], v_t[D_OUT:], b_ntn.reshape(1, K), w_out)
    return out[:, 0]


# speedup vs baseline: 1.0216x; 1.0176x over previous
import jax
import jax.numpy as jnp
from jax.experimental import pallas as pl
from jax.experimental.pallas import tpu as pltpu

B, N, D_IN, D_H, D_OUT, K = 32, 512, 256, 256, 128, 16


def _dot(a, b):
    return jax.lax.dot_general(
        a, b, (((1,), (0,)), ((), ())),
        preferred_element_type=jnp.float32,
    )


PAIRS = 2
STEPS = B // PAIRS


def _gcntn_kernel(x1_ref, x2_ref, l1_ref, l2_ref, w1_ref, w2_ref, wtr_ref,
                  seg_ref, v1t_ref, v2t_ref, b_ref, wo_ref, out_ref, e_ref):
    b = pl.program_id(0)
    w1 = w1_ref[...]
    w2 = w2_ref[...]
    pool = jnp.full((1, N), 1.0 / N, dtype=jnp.float32)

    def tower(x_ref, l_ref, i, row):
        x = x_ref[i]
        l = l_ref[i]
        h = jnp.maximum(_dot(l, _dot(x, w1)), 0.0)
        h = jnp.maximum(_dot(l, _dot(h, w2)), 0.0)
        e_ref[pl.ds(row, 1), :] = _dot(pool, h)

    for i in range(PAIRS):
        tower(x1_ref, l1_ref, i, b * PAIRS + i)
        tower(x2_ref, l2_ref, i, b * PAIRS + i + B)

    @pl.when(b == STEPS - 1)
    def _ntn():
        e1 = e_ref[0:B, :]
        e2 = e_ref[B:2 * B, :]
        t = _dot(e1, wtr_ref[...])
        bil = _dot(t * jnp.tile(e2, (1, K)), seg_ref[...])
        lin = _dot(e1, v1t_ref[...]) + _dot(e2, v2t_ref[...])
        ntn = jnp.maximum(bil + lin + b_ref[...], 0.0)
        out_ref[...] = _dot(ntn, wo_ref[...])


@jax.jit
def kernel(inputs_1, inputs_2, laplacians_1, laplacians_2, W1, W2, Wt, V,
           b_ntn, w_out):
    wt_r = jnp.transpose(Wt, (1, 0, 2)).reshape(D_OUT, K * D_OUT)
    seg = (jnp.arange(K * D_OUT)[:, None] // D_OUT
           == jnp.arange(K)[None, :]).astype(jnp.float32)
    v_t = V.T

    full = lambda *shape: pl.BlockSpec(shape, lambda b: (0,) * len(shape))
    batched = lambda *shape: pl.BlockSpec((PAIRS,) + shape,
                                          lambda b: (b,) + (0,) * len(shape))
    out = pl.pallas_call(
        _gcntn_kernel,
        grid=(STEPS,),
        in_specs=[
            batched(N, D_IN), batched(N, D_IN),
            batched(N, N), batched(N, N),
            full(D_IN, D_H), full(D_H, D_OUT),
            full(D_OUT, K * D_OUT), full(K * D_OUT, K),
            full(D_OUT, K), full(D_OUT, K),
            full(1, K), full(K, 1),
        ],
        out_specs=pl.BlockSpec((B, 1), lambda b: (0, 0)),
        out_shape=jax.ShapeDtypeStruct((B, 1), jnp.float32),
        scratch_shapes=[pltpu.VMEM((2 * B, D_OUT), jnp.float32)],
        compiler_params=pltpu.CompilerParams(
            dimension_semantics=("arbitrary",),
        ),
    )(inputs_1, inputs_2, laplacians_1, laplacians_2, W1, W2, wt_r, seg,
      v_t[:D_OUT], v_t[D_OUT:], b_ntn.reshape(1, K), w_out)
    return out[:, 0]
